# baseline (speedup 1.0000x reference)
import functools

import jax
import jax.numpy as jnp
from jax import lax
from jax.experimental import pallas as pl
from jax.experimental.pallas import tpu as pltpu
from jax.experimental.pallas import tpu_sc as plsc

B = 1024
S = 200
D = 128

NC = 2
NS = 16
NW = NC * NS

SLICES = (256, 256, 512)

CHUNK = 100
CPAD = 128
NCHUNK = S // CHUNK
GROUP = 16

_BLK = 256
_JT = 64


def _rank_body(ts_ref, out_ref, xup_ref, thr_ref, *, col_off):
    bits = lax.bitcast_convert_type(ts_ref[...] + 0.0, jnp.int32)
    bup = bits + jnp.where(bits >= 0, 1, -1)
    xup_ref[...] = lax.bitcast_convert_type(bup, jnp.float32)

    col = lax.broadcasted_iota(jnp.int32, (1, _BLK), 1)
    base = (col_off + pl.program_id(0) * _BLK + col) * S

    for j0 in range(0, S, _JT):
        jt = min(_JT, S - j0)
        x_t = ts_ref[j0:j0 + jt, :]
        xup_t = xup_ref[j0:j0 + jt, :]
        acc0 = jnp.broadcast_to(base, (jt, _BLK)).astype(jnp.int32)

        def s1(k, a):
            rk = ts_ref[pl.ds(k, 1), :]
            return a + (rk < xup_t).astype(jnp.int32)

        acc = lax.fori_loop(0, j0, s1, acc0, unroll=8)

        thr_ref[0:jt, :] = xup_t
        thr_ref[pl.ds(0, 1), :] = ts_ref[pl.ds(j0, 1), :]

        def s2(k, a):
            rk = ts_ref[pl.ds(k, 1), :]
            a = a + (rk < thr_ref[0:jt, :]).astype(jnp.int32)
            kk = jnp.minimum(k + 1, S - 1)
            thr_ref[pl.ds(k - j0 + 1, 1), :] = ts_ref[pl.ds(kk, 1), :]
            return a

        acc = lax.fori_loop(j0, j0 + jt, s2, acc, unroll=2)

        def s3(k, a):
            rk = ts_ref[pl.ds(k, 1), :]
            return a + (rk < x_t).astype(jnp.int32)

        acc = lax.fori_loop(j0 + jt, S, s3, acc, unroll=8)
        out_ref[j0:j0 + jt, :] = acc


def _rank_slice(tsT, col_off, sb):
    blk0 = col_off // _BLK
    return pl.pallas_call(
        functools.partial(_rank_body, col_off=col_off),
        out_shape=jax.ShapeDtypeStruct((S, sb), jnp.int32),
        grid=(sb // _BLK,),
        in_specs=[pl.BlockSpec((S, _BLK), lambda i, b0=blk0: (0, b0 + i))],
        out_specs=pl.BlockSpec((S, _BLK), lambda i: (0, i)),
        scratch_shapes=[
            pltpu.VMEM((S, _BLK), jnp.float32),
            pltpu.VMEM((_JT + 8, _BLK), jnp.float32),
        ],
    )(tsT)


def _scatter_body(pos_hbm, gidx_hbm, out_hbm, pos_v, idx_v, sem, *, cpws, grp):
    wid = lax.axis_index("s") * NC + lax.axis_index("c")
    pltpu.sync_copy(pos_hbm, pos_v)
    pltpu.sync_copy(gidx_hbm.at[wid], idx_v)

    def group(g, carry):
        copies = []
        for i in range(grp):
            src = pos_v.at[pl.ds((i % NCHUNK) * CHUNK, CHUNK)]
            idx = idx_v.at[g * grp + i, pl.ds(0, CHUNK)]
            copies.append(pltpu.async_copy(src, out_hbm.at[idx], sem))
        for c in copies:
            c.wait()
        return carry

    lax.fori_loop(0, cpws // grp, group, 0, unroll=False)


_SC_MESH = plsc.VectorSubcoreMesh(core_axis_name="c", subcore_axis_name="s")


@functools.cache
def _scatter_call(sb, first):
    cpws = (sb // NW) * NCHUNK
    grp = GROUP if cpws % GROUP == 0 else (12 if cpws % 12 == 0 else cpws)
    return pl.kernel(
        functools.partial(_scatter_body, cpws=cpws, grp=grp),
        out_type=jax.ShapeDtypeStruct((B * S, D), jnp.float32) if first else (),
        mesh=_SC_MESH,
        scratch_types=[
            pltpu.VMEM((S, D), jnp.float32),
            pltpu.VMEM((cpws, CPAD), jnp.int32),
            pltpu.SemaphoreType.DMA,
        ],
    )


def _gidx(gT_s, sb):
    g = gT_s.T.reshape(NW, (sb // NW) * NCHUNK, CHUNK)
    return jnp.pad(g, ((0, 0), (0, 0), (0, CPAD - CHUNK)))


def kernel(ts, pos_emb):
    tsT = ts.T
    sb0 = SLICES[0]
    out0 = _scatter_call(sb0, True)(pos_emb, _gidx(_rank_slice(tsT, 0, sb0), sb0))
    if len(SLICES) == 1:
        return out0.reshape(B, S, D)
    oref = jax.new_ref(out0)
    off = sb0
    for sb in SLICES[1:]:
        _scatter_call(sb, False)(pos_emb, _gidx(_rank_slice(tsT, off, sb), sb), oref)
        off += sb
    return oref[...].reshape(B, S, D)


# speedup vs baseline: 1.1636x; 1.1636x over previous
import functools

import jax
import jax.numpy as jnp
from jax import lax
from jax.experimental import pallas as pl
from jax.experimental.pallas import tpu as pltpu
from jax.experimental.pallas import tpu_sc as plsc

B = 1024
S = 200
D = 128

NC = 2
NS = 16
NW = NC * NS

SLICES = (256, 768)

CHUNK = 100
CPAD = 128
NCHUNK = S // CHUNK
GROUP = 16

_BLK = 256
_JT = 72


def _rank_body(ts_ref, out_ref, xup_ref, thr_ref, *, col_off):
    bits = lax.bitcast_convert_type(ts_ref[...] + 0.0, jnp.int32)
    bup = bits + jnp.where(bits >= 0, 1, -1)
    xup_ref[...] = lax.bitcast_convert_type(bup, jnp.float32)

    col = lax.broadcasted_iota(jnp.int32, (1, _BLK), 1)
    base = (col_off + pl.program_id(0) * _BLK + col) * S

    for j0 in range(0, S, _JT):
        jt = min(_JT, S - j0)
        x_t = ts_ref[j0:j0 + jt, :]
        xup_t = xup_ref[j0:j0 + jt, :]
        acc0 = jnp.broadcast_to(base, (jt, _BLK)).astype(jnp.int32)

        def s1(k, a):
            rk = ts_ref[pl.ds(k, 1), :]
            return a + (rk < xup_t).astype(jnp.int32)

        acc = lax.fori_loop(0, j0, s1, acc0, unroll=8)

        thr_ref[0:jt, :] = xup_t
        thr_ref[pl.ds(0, 1), :] = ts_ref[pl.ds(j0, 1), :]

        def s2(k, a):
            rk = ts_ref[pl.ds(k, 1), :]
            a = a + (rk < thr_ref[0:jt, :]).astype(jnp.int32)
            kk = jnp.minimum(k + 1, S - 1)
            thr_ref[pl.ds(k - j0 + 1, 1), :] = ts_ref[pl.ds(kk, 1), :]
            return a

        acc = lax.fori_loop(j0, j0 + jt, s2, acc, unroll=2)

        def s3(k, a):
            rk = ts_ref[pl.ds(k, 1), :]
            return a + (rk < x_t).astype(jnp.int32)

        acc = lax.fori_loop(j0 + jt, S, s3, acc, unroll=8)
        out_ref[j0:j0 + jt, :] = acc


def _rank_slice(tsT, col_off, sb):
    blk0 = col_off // _BLK
    return pl.pallas_call(
        functools.partial(_rank_body, col_off=col_off),
        out_shape=jax.ShapeDtypeStruct((S, sb), jnp.int32),
        grid=(sb // _BLK,),
        in_specs=[pl.BlockSpec((S, _BLK), lambda i, b0=blk0: (0, b0 + i))],
        out_specs=pl.BlockSpec((S, _BLK), lambda i: (0, i)),
        scratch_shapes=[
            pltpu.VMEM((S, _BLK), jnp.float32),
            pltpu.VMEM((_JT + 8, _BLK), jnp.float32),
        ],
    )(tsT)


def _scatter_body(pos_hbm, gidx_hbm, out_hbm, pos_v, idx_v, sem, *, cpws, grp):
    wid = lax.axis_index("s") * NC + lax.axis_index("c")
    pltpu.sync_copy(pos_hbm, pos_v)
    pltpu.sync_copy(gidx_hbm.at[wid], idx_v)

    def group(g, carry):
        copies = []
        for i in range(grp):
            src = pos_v.at[pl.ds((i % NCHUNK) * CHUNK, CHUNK)]
            idx = idx_v.at[g * grp + i, pl.ds(0, CHUNK)]
            copies.append(pltpu.async_copy(src, out_hbm.at[idx], sem))
        for c in copies:
            c.wait()
        return carry

    lax.fori_loop(0, cpws // grp, group, 0, unroll=False)


_SC_MESH = plsc.VectorSubcoreMesh(core_axis_name="c", subcore_axis_name="s")


@functools.cache
def _scatter_call(sb, first):
    cpws = (sb // NW) * NCHUNK
    grp = GROUP if cpws % GROUP == 0 else (12 if cpws % 12 == 0 else cpws)
    return pl.kernel(
        functools.partial(_scatter_body, cpws=cpws, grp=grp),
        out_type=jax.ShapeDtypeStruct((B * S, D), jnp.float32) if first else (),
        mesh=_SC_MESH,
        scratch_types=[
            pltpu.VMEM((S, D), jnp.float32),
            pltpu.VMEM((cpws, CPAD), jnp.int32),
            pltpu.SemaphoreType.DMA,
        ],
    )


def _gidx(gT_s, sb):
    g = gT_s.T.reshape(NW, (sb // NW) * NCHUNK, CHUNK)
    return jnp.pad(g, ((0, 0), (0, 0), (0, CPAD - CHUNK)))


def kernel(ts, pos_emb):
    tsT = ts.T
    sb0 = SLICES[0]
    out0 = _scatter_call(sb0, True)(pos_emb, _gidx(_rank_slice(tsT, 0, sb0), sb0))
    if len(SLICES) == 1:
        return out0.reshape(B, S, D)
    oref = jax.new_ref(out0)
    off = sb0
    for sb in SLICES[1:]:
        _scatter_call(sb, False)(pos_emb, _gidx(_rank_slice(tsT, off, sb), sb), oref)
        off += sb
    return oref[...].reshape(B, S, D)


# speedup vs baseline: 1.1815x; 1.0155x over previous
import functools

import jax
import jax.numpy as jnp
from jax import lax
from jax.experimental import pallas as pl
from jax.experimental.pallas import tpu as pltpu
from jax.experimental.pallas import tpu_sc as plsc

B = 1024
S = 200
D = 128

NC = 2
NS = 16
NW = NC * NS

SLICES = (256, 768)

CHUNK = 100
CPAD = 128
NCHUNK = S // CHUNK
GROUP = 16

_BLK = 256
_JT = 72


def _rank_body(ts_ref, out_ref, xup_ref, thr_ref, *, col_off):
    bits = lax.bitcast_convert_type(ts_ref[...] + 0.0, jnp.int32)
    bup = bits + jnp.where(bits >= 0, 1, -1)
    xup_ref[...] = lax.bitcast_convert_type(bup, jnp.float32)

    col = lax.broadcasted_iota(jnp.int32, (1, _BLK), 1)
    base = (col_off + pl.program_id(0) * _BLK + col) * S

    for j0 in range(0, S, _JT):
        jt = min(_JT, S - j0)
        x_t = ts_ref[j0:j0 + jt, :]
        xup_t = xup_ref[j0:j0 + jt, :]
        acc0 = jnp.broadcast_to(base, (jt, _BLK)).astype(jnp.int32)

        def s1(k, a):
            rk = ts_ref[pl.ds(k, 1), :]
            return a + (rk < xup_t).astype(jnp.int32)

        acc = lax.fori_loop(0, j0, s1, acc0, unroll=8)

        thr_ref[0:jt, :] = xup_t
        thr_ref[pl.ds(0, 1), :] = ts_ref[pl.ds(j0, 1), :]

        def s2(k, a):
            rk = ts_ref[pl.ds(k, 1), :]
            a = a + (rk < thr_ref[0:jt, :]).astype(jnp.int32)
            kk = jnp.minimum(k + 1, S - 1)
            thr_ref[pl.ds(k - j0 + 1, 1), :] = ts_ref[pl.ds(kk, 1), :]
            return a

        acc = lax.fori_loop(j0, j0 + jt, s2, acc, unroll=4)

        def s3(k, a):
            rk = ts_ref[pl.ds(k, 1), :]
            return a + (rk < x_t).astype(jnp.int32)

        acc = lax.fori_loop(j0 + jt, S, s3, acc, unroll=8)
        out_ref[j0:j0 + jt, :] = acc


def _rank_slice(tsT, col_off, sb):
    blk0 = col_off // _BLK
    return pl.pallas_call(
        functools.partial(_rank_body, col_off=col_off),
        out_shape=jax.ShapeDtypeStruct((S, sb), jnp.int32),
        grid=(sb // _BLK,),
        in_specs=[pl.BlockSpec((S, _BLK), lambda i, b0=blk0: (0, b0 + i))],
        out_specs=pl.BlockSpec((S, _BLK), lambda i: (0, i)),
        scratch_shapes=[
            pltpu.VMEM((S, _BLK), jnp.float32),
            pltpu.VMEM((_JT + 8, _BLK), jnp.float32),
        ],
    )(tsT)


def _scatter_body(pos_hbm, gidx_hbm, out_hbm, pos_v, idx_v, sem, *, cpws, grp):
    wid = lax.axis_index("s") * NC + lax.axis_index("c")
    pltpu.sync_copy(pos_hbm, pos_v)
    pltpu.sync_copy(gidx_hbm.at[wid], idx_v)

    def group(g, carry):
        copies = []
        for i in range(grp):
            src = pos_v.at[pl.ds((i % NCHUNK) * CHUNK, CHUNK)]
            idx = idx_v.at[g * grp + i, pl.ds(0, CHUNK)]
            copies.append(pltpu.async_copy(src, out_hbm.at[idx], sem))
        for c in copies:
            c.wait()
        return carry

    lax.fori_loop(0, cpws // grp, group, 0, unroll=False)


_SC_MESH = plsc.VectorSubcoreMesh(core_axis_name="c", subcore_axis_name="s")


@functools.cache
def _scatter_call(sb, first):
    cpws = (sb // NW) * NCHUNK
    grp = GROUP if cpws % GROUP == 0 else (12 if cpws % 12 == 0 else cpws)
    return pl.kernel(
        functools.partial(_scatter_body, cpws=cpws, grp=grp),
        out_type=jax.ShapeDtypeStruct((B * S, D), jnp.float32) if first else (),
        mesh=_SC_MESH,
        scratch_types=[
            pltpu.VMEM((S, D), jnp.float32),
            pltpu.VMEM((cpws, CPAD), jnp.int32),
            pltpu.SemaphoreType.DMA,
        ],
    )


def _gidx(gT_s, sb):
    g = gT_s.T.reshape(NW, (sb // NW) * NCHUNK, CHUNK)
    return jnp.pad(g, ((0, 0), (0, 0), (0, CPAD - CHUNK)))


def kernel(ts, pos_emb):
    tsT = ts.T
    sb0 = SLICES[0]
    out0 = _scatter_call(sb0, True)(pos_emb, _gidx(_rank_slice(tsT, 0, sb0), sb0))
    if len(SLICES) == 1:
        return out0.reshape(B, S, D)
    oref = jax.new_ref(out0)
    off = sb0
    for sb in SLICES[1:]:
        _scatter_call(sb, False)(pos_emb, _gidx(_rank_slice(tsT, off, sb), sb), oref)
        off += sb
    return oref[...].reshape(B, S, D)
